# baseline (speedup 1.0000x reference)
import functools

import jax
import jax.numpy as jnp
from jax import lax
from jax.experimental import pallas as pl
from jax.experimental.pallas import tpu as pltpu
from jax.experimental.pallas import tpu_sc as plsc

NC = 2
NS = 16
L = 16

SPAD = 208
NG = SPAD // L
CH = 104


def kernel(embed, shortlist, sp_weight, sp_bias):
    B, S = shortlist.shape
    H = embed.shape[1]
    short_flat = shortlist.astype(jnp.int32).reshape(-1)
    embed_flat = embed.reshape(-1)
    bias_flat = sp_bias.reshape(-1)

    NW = NC * NS
    BPW = B // NW

    mesh = plsc.VectorSubcoreMesh(core_axis_name="c", subcore_axis_name="s")

    @functools.partial(
        pl.kernel,
        out_type=jax.ShapeDtypeStruct((B * S,), jnp.float32),
        mesh=mesh,
        compiler_params=pltpu.CompilerParams(
            needs_layout_passes=False, use_tc_tiling_on_sc=False),
        scratch_types=[
            pltpu.VMEM((2, CH), jnp.int32),
            pltpu.VMEM((SPAD, H), jnp.float32),
            pltpu.VMEM((SPAD,), jnp.float32),
            pltpu.VMEM((H,), jnp.float32),
            pltpu.VMEM((SPAD,), jnp.float32),
            pltpu.SemaphoreType.DMA,
        ],
    )
    def run(embed_hbm, short_hbm, table_hbm, bias_hbm, out_hbm,
            idx2, rows_v, bias_v, emb_v, out_v, sem):
        wid = lax.axis_index("s") * NC + lax.axis_index("c")
        base = wid * BPW
        iota = lax.iota(jnp.int32, L)
        svecs = [jnp.int32(g * L) + iota for g in range(NG)]

        def body(i, carry):
            b = base + i
            pltpu.sync_copy(short_hbm.at[pl.ds(b * S, CH)], idx2.at[0])
            pltpu.sync_copy(short_hbm.at[pl.ds(b * S + CH, S - CH)],
                            idx2.at[1, pl.ds(0, S - CH)])
            tail = idx2[1, pl.ds(S - CH - 8, L)]
            idx2[1, pl.ds(S - CH - 8, L)] = jnp.where(iota < 8, tail, 0)
            pltpu.sync_copy(embed_hbm.at[pl.ds(b * H, H)], emb_v)
            cps = [
                pltpu.async_copy(table_hbm.at[idx2.at[0]],
                                 rows_v.at[pl.ds(0, CH)], sem),
                pltpu.async_copy(table_hbm.at[idx2.at[1]],
                                 rows_v.at[pl.ds(CH, CH)], sem),
                pltpu.async_copy(bias_hbm.at[idx2.at[0]],
                                 bias_v.at[pl.ds(0, CH)], sem),
                pltpu.async_copy(bias_hbm.at[idx2.at[1]],
                                 bias_v.at[pl.ds(CH, CH)], sem),
            ]
            for cp in cps:
                cp.wait()

            accs = tuple(bias_v[pl.ds(g * L, L)] for g in range(NG))

            def hbody(h, accs):
                hv = jnp.full((L,), h, dtype=jnp.int32)
                eb = plsc.load_gather(emb_v, [hv])
                return tuple(
                    acc + plsc.load_gather(rows_v, [sv, hv]) * eb
                    for acc, sv in zip(accs, svecs)
                )

            accs = lax.fori_loop(0, H, hbody, accs)
            for g in range(NG):
                out_v[pl.ds(g * L, L)] = accs[g]
            pltpu.sync_copy(out_v.at[pl.ds(0, S)],
                            out_hbm.at[pl.ds(b * S, S)])
            return carry

        lax.fori_loop(0, BPW, body, 0)

    out = run(embed_flat, short_flat, sp_weight, bias_flat)
    return out.reshape(B, S)
